# baseline (speedup 1.0000x reference)
import functools

import jax
import jax.numpy as jnp
from jax import lax
from jax.experimental import pallas as pl
from jax.experimental.pallas import tpu as pltpu

(_G, _STD_IN, _STD_OUT, _ALPHA, _RHO, _K1, _K2, _K3,
 _V, _E0, _TAU_S, _TAU_F, _TAU_0, _DT, _SQRT_DT) = range(15)
_NUM_PARAMS = 16


def _lap_kernel(params_ref, sc_ref, gc_ref, lap_ref):
    g = params_ref[_G]
    sc_mod = jnp.exp(gc_ref[...]) * sc_ref[...]
    sc_sym = 0.5 * (sc_mod + sc_mod.T)
    fro = jnp.sqrt(jnp.sum(sc_sym * sc_sym))
    sc_n = sc_sym / fro
    row_sum = jnp.sum(sc_n, axis=1, keepdims=True)
    rr = lax.broadcasted_iota(jnp.int32, sc_n.shape, 0)
    cc = lax.broadcasted_iota(jnp.int32, sc_n.shape, 1)
    lap = jnp.where(rr == cc, sc_n - row_sum, sc_n)
    lap_ref[...] = g * lap


def _sim_kernel(params_ref, lap_ref, hx_ref, ext_ref, ne_ref, nb_ref,
                swin_ref, bold_ref, cur_ref, st_ref,
                *, steps_per_tr, trs_per_window):
    t = pl.program_id(1)

    std_in  = params_ref[_STD_IN]
    std_out = params_ref[_STD_OUT]
    alpha   = params_ref[_ALPHA]
    rho     = params_ref[_RHO]
    k1      = params_ref[_K1]
    k2      = params_ref[_K2]
    k3      = params_ref[_K3]
    V       = params_ref[_V]
    E0      = params_ref[_E0]
    tau_s   = params_ref[_TAU_S]
    tau_f   = params_ref[_TAU_F]
    tau_0   = params_ref[_TAU_0]
    dt      = params_ref[_DT]
    sqrt_dt = params_ref[_SQRT_DT]

    inv_alpha    = 1.0 / alpha
    inv_alpha_m1 = inv_alpha - 1.0
    inv_rho      = 1.0 / rho
    inv_tau_s    = 1.0 / tau_s
    inv_tau_f    = 1.0 / tau_f
    dt_tau0      = dt / tau_0
    log1m_rho    = jnp.log(1.0 - rho)
    noise_scale  = sqrt_dt * (0.1 + std_in)
    bold_gain    = 100.0 * V / E0

    @pl.when(t == 0)
    def _init():
        st_ref[...] = hx_ref[...]

    E = st_ref[0]
    x = st_ref[1]
    f = st_ref[2]
    v = st_ref[3]
    q = st_ref[4]
    lap_g = lap_ref[...].astype(jnp.bfloat16)

    for s in range(steps_per_tr):
        u  = ext_ref[0, s:s + 1, :]
        nE = ne_ref[0, s]

        IE = jnp.dot(E.astype(jnp.bfloat16), lap_g,
                     preferred_element_type=jnp.float32) + u

        lv        = jnp.log(v)
        v_pow     = jnp.exp(inv_alpha * lv)
        v_pow_dv  = jnp.exp(inv_alpha_m1 * lv)
        pow_rho_f = jnp.exp(log1m_rho * pl.reciprocal(f, approx=True))

        E_next = E + dt * (-E + jnp.tanh(IE)) + noise_scale * nE
        x_next = x + dt * (E - x * inv_tau_s - (f - 1.0) * inv_tau_f)
        f_next = f + dt * x
        v_next = v + dt_tau0 * (f - v_pow)
        q_next = q + dt_tau0 * (f * (1.0 - pow_rho_f) * inv_rho - q * v_pow_dv)

        E = jnp.tanh(E_next)
        x = x_next
        f = 1.0 + jnp.tanh(f_next - 1.0)
        v = 1.0 + jnp.tanh(v_next - 1.0)
        q = 1.0 + jnp.tanh(q_next - 1.0)

    st_ref[0] = E
    st_ref[1] = x
    st_ref[2] = f
    st_ref[3] = v
    st_ref[4] = q

    swin_ref[0, 0] = E
    swin_ref[1, 0] = x
    swin_ref[2, 0] = f
    swin_ref[3, 0] = v
    swin_ref[4, 0] = q

    bold_ref[0] = (std_out * nb_ref[0]
                   + bold_gain * (k1 * (1.0 - q)
                                  + k2 * (1.0 - q / v)
                                  + k3 * (1.0 - v)))

    @pl.when(t == trs_per_window - 1)
    def _finalize():
        cur_ref[...] = st_ref[...]


def kernel(external, hx_batch, hE, sc, gains_con, g, std_in, std_out, alpha,
           rho, k1, k2, k3, V, E0, tau_s, tau_f, tau_0, noise_seed):
    step_size = 0.05
    tr = 0.75
    S = int(tr / step_size)
    N, _, T = external.shape
    B = hx_batch.shape[0]
    TS = T * S

    nb = 1
    b_block = B // nb

    p = jnp.zeros((_NUM_PARAMS,), jnp.float32)
    p = p.at[:15].set(jnp.array(
        [g, std_in, std_out, alpha, rho, k1, k2, k3, V, E0,
         tau_s, tau_f, tau_0, step_size, step_size ** 0.5], jnp.float32))

    noise_key = jax.random.wrap_key_data(noise_seed)
    k_e, k_b = jax.random.split(noise_key)
    noise_e = jax.random.normal(k_e, (TS, B, N), jnp.float32).reshape(T, S, B, N)
    noise_b = jax.random.normal(k_b, (T, B, N), jnp.float32)

    ext = jnp.transpose(external.astype(jnp.float32), (2, 1, 0))
    hx_sbn = jnp.transpose(hx_batch.astype(jnp.float32), (2, 0, 1))
    sc32 = sc.astype(jnp.float32)
    gc32 = gains_con.astype(jnp.float32)

    lap_g = pl.pallas_call(
        _lap_kernel,
        out_shape=jax.ShapeDtypeStruct((N, N), jnp.float32),
        grid=(1,),
        in_specs=[
            pl.BlockSpec((_NUM_PARAMS,), lambda i: (0,),
                         memory_space=pltpu.MemorySpace.SMEM),
            pl.BlockSpec((N, N), lambda i: (0, 0)),
            pl.BlockSpec((N, N), lambda i: (0, 0)),
        ],
        out_specs=pl.BlockSpec((N, N), lambda i: (0, 0)),
    )(p, sc32, gc32)

    _kernel_fn = functools.partial(_sim_kernel,
                                   steps_per_tr=S, trs_per_window=T)

    in_specs = [
        pl.BlockSpec((_NUM_PARAMS,), lambda b, t: (0,),
                     memory_space=pltpu.MemorySpace.SMEM),
        pl.BlockSpec((N, N), lambda b, t: (0, 0)),
        pl.BlockSpec((5, b_block, N), lambda b, t: (0, b, 0)),
        pl.BlockSpec((1, S, N), lambda b, t: (t, 0, 0)),
        pl.BlockSpec((1, S, b_block, N), lambda b, t: (t, 0, b, 0)),
        pl.BlockSpec((1, b_block, N), lambda b, t: (t, b, 0)),
    ]
    out_specs = [
        pl.BlockSpec((5, 1, b_block, N), lambda b, t: (0, t, b, 0)),
        pl.BlockSpec((1, b_block, N), lambda b, t: (t, b, 0)),
        pl.BlockSpec((5, b_block, N), lambda b, t: (0, b, 0)),
    ]
    out_shapes = (
        jax.ShapeDtypeStruct((5, T, B, N), jnp.float32),
        jax.ShapeDtypeStruct((T, B, N), jnp.float32),
        jax.ShapeDtypeStruct((5, B, N), jnp.float32),
    )

    state_win, bold_win, cur = pl.pallas_call(
        _kernel_fn,
        out_shape=out_shapes,
        grid=(nb, T),
        in_specs=in_specs,
        out_specs=out_specs,
        scratch_shapes=[pltpu.VMEM((5, b_block, N), jnp.float32)],
        compiler_params=pltpu.CompilerParams(
            dimension_semantics=("parallel", "arbitrary")),
    )(p, lap_g, hx_sbn, ext, noise_e, noise_b)

    next_state = {
        "current_state": jnp.transpose(cur, (1, 2, 0)),
        "bold_window":   jnp.transpose(bold_win, (1, 2, 0)),
        "E_window":      jnp.transpose(state_win[0], (1, 2, 0)),
        "x_window":      jnp.transpose(state_win[1], (1, 2, 0)),
        "f_window":      jnp.transpose(state_win[2], (1, 2, 0)),
        "v_window":      jnp.transpose(state_win[3], (1, 2, 0)),
        "q_window":      jnp.transpose(state_win[4], (1, 2, 0)),
    }
    return next_state, hE


# speedup vs baseline: 1.0050x; 1.0050x over previous
import functools

import jax
import jax.numpy as jnp
from jax import lax
from jax.experimental import pallas as pl
from jax.experimental.pallas import tpu as pltpu

(_G, _STD_IN, _STD_OUT, _ALPHA, _RHO, _K1, _K2, _K3,
 _V, _E0, _TAU_S, _TAU_F, _TAU_0, _DT, _SQRT_DT) = range(15)
_NUM_PARAMS = 16


def _lap_kernel(params_ref, sc_ref, gc_ref, lap_ref):
    g = params_ref[_G]
    sc_mod = jnp.exp(gc_ref[...]) * sc_ref[...]
    sc_sym = 0.5 * (sc_mod + sc_mod.T)
    fro = jnp.sqrt(jnp.sum(sc_sym * sc_sym))
    sc_n = sc_sym / fro
    row_sum = jnp.sum(sc_n, axis=1, keepdims=True)
    rr = lax.broadcasted_iota(jnp.int32, sc_n.shape, 0)
    cc = lax.broadcasted_iota(jnp.int32, sc_n.shape, 1)
    lap = jnp.where(rr == cc, sc_n - row_sum, sc_n)
    lap_ref[...] = g * lap


def _sim_kernel(params_ref, lap_ref, hx_ref, ext_ref, ne_ref, nb_ref,
                swin_ref, bold_ref, cur_ref, st_ref,
                *, steps_per_tr, trs_per_window):
    t = pl.program_id(1)

    std_in  = params_ref[_STD_IN]
    std_out = params_ref[_STD_OUT]
    alpha   = params_ref[_ALPHA]
    rho     = params_ref[_RHO]
    k1      = params_ref[_K1]
    k2      = params_ref[_K2]
    k3      = params_ref[_K3]
    V       = params_ref[_V]
    E0      = params_ref[_E0]
    tau_s   = params_ref[_TAU_S]
    tau_f   = params_ref[_TAU_F]
    tau_0   = params_ref[_TAU_0]
    dt      = params_ref[_DT]
    sqrt_dt = params_ref[_SQRT_DT]

    inv_alpha    = 1.0 / alpha
    inv_alpha_m1 = inv_alpha - 1.0
    inv_rho      = 1.0 / rho
    inv_tau_s    = 1.0 / tau_s
    inv_tau_f    = 1.0 / tau_f
    dt_tau0      = dt / tau_0
    log1m_rho    = jnp.log(1.0 - rho)
    noise_scale  = sqrt_dt * (0.1 + std_in)
    bold_gain    = 100.0 * V / E0

    @pl.when(t == 0)
    def _init():
        st_ref[...] = hx_ref[...]

    E = st_ref[0]
    x = st_ref[1]
    f = st_ref[2]
    v = st_ref[3]
    q = st_ref[4]
    lap_g = lap_ref[...]

    for s in range(steps_per_tr):
        u  = ext_ref[0, s:s + 1, :]
        nE = ne_ref[0, s]

        IE = jnp.dot(E, lap_g, preferred_element_type=jnp.float32) + u

        lv        = jnp.log(v)
        v_pow     = jnp.exp(inv_alpha * lv)
        v_pow_dv  = jnp.exp(inv_alpha_m1 * lv)
        pow_rho_f = jnp.exp(log1m_rho * pl.reciprocal(f, approx=True))

        E_next = E + dt * (-E + jnp.tanh(IE)) + noise_scale * nE
        x_next = x + dt * (E - x * inv_tau_s - (f - 1.0) * inv_tau_f)
        f_next = f + dt * x
        v_next = v + dt_tau0 * (f - v_pow)
        q_next = q + dt_tau0 * (f * (1.0 - pow_rho_f) * inv_rho - q * v_pow_dv)

        E = jnp.tanh(E_next)
        x = x_next
        f = 1.0 + jnp.tanh(f_next - 1.0)
        v = 1.0 + jnp.tanh(v_next - 1.0)
        q = 1.0 + jnp.tanh(q_next - 1.0)

    st_ref[0] = E
    st_ref[1] = x
    st_ref[2] = f
    st_ref[3] = v
    st_ref[4] = q

    swin_ref[0, 0] = E
    swin_ref[1, 0] = x
    swin_ref[2, 0] = f
    swin_ref[3, 0] = v
    swin_ref[4, 0] = q

    bold_ref[0] = (std_out * nb_ref[0]
                   + bold_gain * (k1 * (1.0 - q)
                                  + k2 * (1.0 - q / v)
                                  + k3 * (1.0 - v)))

    @pl.when(t == trs_per_window - 1)
    def _finalize():
        cur_ref[...] = st_ref[...]


def kernel(external, hx_batch, hE, sc, gains_con, g, std_in, std_out, alpha,
           rho, k1, k2, k3, V, E0, tau_s, tau_f, tau_0, noise_seed):
    step_size = 0.05
    tr = 0.75
    S = int(tr / step_size)
    N, _, T = external.shape
    B = hx_batch.shape[0]
    TS = T * S

    nb = 1
    b_block = B // nb

    p = jnp.zeros((_NUM_PARAMS,), jnp.float32)
    p = p.at[:15].set(jnp.array(
        [g, std_in, std_out, alpha, rho, k1, k2, k3, V, E0,
         tau_s, tau_f, tau_0, step_size, step_size ** 0.5], jnp.float32))

    noise_key = jax.random.wrap_key_data(noise_seed)
    k_e, k_b = jax.random.split(noise_key)
    noise_e = jax.random.normal(k_e, (TS, B, N), jnp.float32).reshape(T, S, B, N)
    noise_b = jax.random.normal(k_b, (T, B, N), jnp.float32)

    ext = jnp.transpose(external.astype(jnp.float32), (2, 1, 0))
    hx_sbn = jnp.transpose(hx_batch.astype(jnp.float32), (2, 0, 1))
    sc32 = sc.astype(jnp.float32)
    gc32 = gains_con.astype(jnp.float32)

    lap_g = pl.pallas_call(
        _lap_kernel,
        out_shape=jax.ShapeDtypeStruct((N, N), jnp.float32),
        grid=(1,),
        in_specs=[
            pl.BlockSpec((_NUM_PARAMS,), lambda i: (0,),
                         memory_space=pltpu.MemorySpace.SMEM),
            pl.BlockSpec((N, N), lambda i: (0, 0)),
            pl.BlockSpec((N, N), lambda i: (0, 0)),
        ],
        out_specs=pl.BlockSpec((N, N), lambda i: (0, 0)),
    )(p, sc32, gc32)

    _kernel_fn = functools.partial(_sim_kernel,
                                   steps_per_tr=S, trs_per_window=T)

    in_specs = [
        pl.BlockSpec((_NUM_PARAMS,), lambda b, t: (0,),
                     memory_space=pltpu.MemorySpace.SMEM),
        pl.BlockSpec((N, N), lambda b, t: (0, 0)),
        pl.BlockSpec((5, b_block, N), lambda b, t: (0, b, 0)),
        pl.BlockSpec((1, S, N), lambda b, t: (t, 0, 0)),
        pl.BlockSpec((1, S, b_block, N), lambda b, t: (t, 0, b, 0)),
        pl.BlockSpec((1, b_block, N), lambda b, t: (t, b, 0)),
    ]
    out_specs = [
        pl.BlockSpec((5, 1, b_block, N), lambda b, t: (0, t, b, 0)),
        pl.BlockSpec((1, b_block, N), lambda b, t: (t, b, 0)),
        pl.BlockSpec((5, b_block, N), lambda b, t: (0, b, 0)),
    ]
    out_shapes = (
        jax.ShapeDtypeStruct((5, T, B, N), jnp.float32),
        jax.ShapeDtypeStruct((T, B, N), jnp.float32),
        jax.ShapeDtypeStruct((5, B, N), jnp.float32),
    )

    state_win, bold_win, cur = pl.pallas_call(
        _kernel_fn,
        out_shape=out_shapes,
        grid=(nb, T),
        in_specs=in_specs,
        out_specs=out_specs,
        scratch_shapes=[pltpu.VMEM((5, b_block, N), jnp.float32)],
        compiler_params=pltpu.CompilerParams(
            dimension_semantics=("parallel", "arbitrary")),
    )(p, lap_g, hx_sbn, ext, noise_e, noise_b)

    next_state = {
        "current_state": jnp.transpose(cur, (1, 2, 0)),
        "bold_window":   jnp.transpose(bold_win, (1, 2, 0)),
        "E_window":      jnp.transpose(state_win[0], (1, 2, 0)),
        "x_window":      jnp.transpose(state_win[1], (1, 2, 0)),
        "f_window":      jnp.transpose(state_win[2], (1, 2, 0)),
        "v_window":      jnp.transpose(state_win[3], (1, 2, 0)),
        "q_window":      jnp.transpose(state_win[4], (1, 2, 0)),
    }
    return next_state, hE
